# Initial kernel scaffold; baseline (speedup 1.0000x reference)
import jax
import jax.numpy as jnp
from jax.experimental import pallas as pl

_B = 4096
_V = 1000
_BR = 512


def _body(out_ref, labels_ref, index_ref, diff_ref, pred_ref, acc_ref):
    b = pl.program_id(0)
    x = out_ref[:, 0, :]
    m = jnp.max(x, axis=1, keepdims=True)
    s = jnp.sum(jnp.exp(x - m), axis=1, keepdims=True)
    lse = m + jnp.log(s)

    iot = jax.lax.broadcasted_iota(jnp.int32, (_BR, _V), 1)
    cols = []
    for i in range(8):
        li = labels_ref[:, 8 + i : 9 + i]
        gi = jnp.sum(jnp.where(iot == li, x, 0.0), axis=1, keepdims=True)
        cols.append(lse - gi)
    d = jnp.concatenate(cols, axis=1)
    diff_ref[:, :] = d

    col = jax.lax.broadcasted_iota(jnp.int32, (_BR, 8), 1)
    mn = jnp.min(d, axis=1, keepdims=True)
    pidx = jnp.min(jnp.where(d == mn, col, 8), axis=1, keepdims=True)
    pred_ref[:, :] = pidx

    match = (index_ref[:, 0:1] - 8) == pidx
    cnt = jnp.sum(match.astype(jnp.float32))

    @pl.when(b == 0)
    def _init():
        acc_ref[0, 0] = 0.0

    acc_ref[0, 0] += cnt

    @pl.when(b == pl.num_programs(0) - 1)
    def _final():
        acc_ref[0, 0] = acc_ref[0, 0] / _B


def kernel(output, labels, index):
    grid = _B // _BR
    diff, pred, acc = pl.pallas_call(
        _body,
        grid=(grid,),
        in_specs=[
            pl.BlockSpec((_BR, 1, _V), lambda b: (b, 7, 0)),
            pl.BlockSpec((_BR, 16), lambda b: (b, 0)),
            pl.BlockSpec((_BR, 2), lambda b: (b, 0)),
        ],
        out_specs=[
            pl.BlockSpec((_BR, 8), lambda b: (b, 0)),
            pl.BlockSpec((_BR, 1), lambda b: (b, 0)),
            pl.BlockSpec((1, 1), lambda b: (0, 0)),
        ],
        out_shape=[
            jax.ShapeDtypeStruct((_B, 8), jnp.float32),
            jax.ShapeDtypeStruct((_B, 1), jnp.int32),
            jax.ShapeDtypeStruct((1, 1), jnp.float32),
        ],
    )(output, labels, index)
    return diff, pred.reshape(_B), acc[0, 0]


# speedup vs baseline: 3.8771x; 3.8771x over previous
import jax
import jax.numpy as jnp
from jax.experimental import pallas as pl
from jax.experimental.pallas import tpu as pltpu

_B = 4096
_V = 1000
_BR = 512


def _body(out_hbm, labels_ref, index_ref, diff_ref, pred_ref, acc_ref,
          xbuf, sems):
    b = pl.program_id(0)
    nb = pl.num_programs(0)

    def copy(step, slot):
        return pltpu.make_async_copy(
            out_hbm.at[pl.ds(step * _BR, _BR), 7, :],
            xbuf.at[slot],
            sems.at[slot],
        )

    @pl.when(b == 0)
    def _prime():
        copy(0, 0).start()

    @pl.when(b + 1 < nb)
    def _next():
        copy(b + 1, (b + 1) % 2).start()

    copy(b, b % 2).wait()
    x = xbuf[b % 2]

    m = jnp.max(x, axis=1, keepdims=True)
    s = jnp.sum(jnp.exp(x - m), axis=1, keepdims=True)
    lse = m + jnp.log(s)

    iot = jax.lax.broadcasted_iota(jnp.int32, (_BR, _V), 1)
    cols = []
    for i in range(8):
        li = labels_ref[:, 8 + i : 9 + i]
        gi = jnp.sum(jnp.where(iot == li, x, 0.0), axis=1, keepdims=True)
        cols.append(lse - gi)
    d = jnp.concatenate(cols, axis=1)
    diff_ref[:, :] = d

    col = jax.lax.broadcasted_iota(jnp.int32, (_BR, 8), 1)
    mn = jnp.min(d, axis=1, keepdims=True)
    pidx = jnp.min(jnp.where(d == mn, col, 8), axis=1, keepdims=True)
    pred_ref[:, :] = pidx

    match = (index_ref[:, 0:1] - 8) == pidx
    cnt = jnp.sum(match.astype(jnp.float32))

    @pl.when(b == 0)
    def _init():
        acc_ref[0, 0] = 0.0

    acc_ref[0, 0] += cnt

    @pl.when(b == nb - 1)
    def _final():
        acc_ref[0, 0] = acc_ref[0, 0] / _B


def kernel(output, labels, index):
    grid = _B // _BR
    diff, pred, acc = pl.pallas_call(
        _body,
        grid=(grid,),
        in_specs=[
            pl.BlockSpec(memory_space=pl.ANY),
            pl.BlockSpec((_BR, 16), lambda b: (b, 0)),
            pl.BlockSpec((_BR, 2), lambda b: (b, 0)),
        ],
        out_specs=[
            pl.BlockSpec((_BR, 8), lambda b: (b, 0)),
            pl.BlockSpec((_BR, 1), lambda b: (b, 0)),
            pl.BlockSpec((1, 1), lambda b: (0, 0), memory_space=pltpu.SMEM),
        ],
        out_shape=[
            jax.ShapeDtypeStruct((_B, 8), jnp.float32),
            jax.ShapeDtypeStruct((_B, 1), jnp.int32),
            jax.ShapeDtypeStruct((1, 1), jnp.float32),
        ],
        scratch_shapes=[
            pltpu.VMEM((2, _BR, _V), jnp.float32),
            pltpu.SemaphoreType.DMA((2,)),
        ],
    )(output, labels, index)
    return diff, pred.reshape(_B), acc[0, 0]
